# baseline (speedup 1.0000x reference)
import functools

import jax
import jax.numpy as jnp
from jax import lax
from jax.experimental import pallas as pl
from jax.experimental.pallas import tpu as pltpu

D_IN = 768
D_OUT = 768
E = 8
T = 2048


def _moe_body(x_ref, wr_ref, br_ref, we_ref, be_ref, out_ref, mask_ref):
    e = pl.program_id(0)

    @pl.when(e == 0)
    def _router():
        logits = lax.dot_general(
            x_ref[...], wr_ref[...], (((1,), (1,)), ((), ())),
            preferred_element_type=jnp.float32,
        ) + br_ref[...]
        i1 = jnp.argmax(logits, axis=1)
        eids = lax.broadcasted_iota(jnp.int32, logits.shape, 1)
        m1 = eids == i1[:, None]
        l2 = jnp.where(m1, -jnp.inf, logits)
        i2 = jnp.argmax(l2, axis=1)
        m2 = eids == i2[:, None]
        mask_ref[...] = (m1 | m2).astype(jnp.float32)

    m = mask_ref[...]
    sel = (lax.broadcasted_iota(jnp.int32, m.shape, 1) == e).astype(jnp.float32)
    col = jnp.sum(m * sel, axis=1, keepdims=True)
    contrib = lax.dot_general(
        x_ref[...], we_ref[0], (((1,), (1,)), ((), ())),
        preferred_element_type=jnp.float32,
    ) + be_ref[0]
    contrib = col * contrib

    @pl.when(e == 0)
    def _init():
        out_ref[...] = contrib

    @pl.when(e != 0)
    def _acc():
        out_ref[...] += contrib


@jax.jit
def _moe(xf, Wr, br2, We, be3):
    return pl.pallas_call(
        _moe_body,
        grid=(E,),
        in_specs=[
            pl.BlockSpec((T, D_IN), lambda e: (0, 0)),
            pl.BlockSpec((E, D_IN), lambda e: (0, 0)),
            pl.BlockSpec((1, E), lambda e: (0, 0)),
            pl.BlockSpec((1, D_OUT, D_IN), lambda e: (e, 0, 0)),
            pl.BlockSpec((1, 1, D_OUT), lambda e: (e, 0, 0)),
        ],
        out_specs=pl.BlockSpec((T, D_OUT), lambda e: (0, 0)),
        out_shape=jax.ShapeDtypeStruct((T, D_OUT), jnp.float32),
        scratch_shapes=[pltpu.VMEM((T, E), jnp.float32)],
    )(xf, Wr, br2, We, be3)


def kernel(x, Wr, br, We, be):
    xf = x.reshape(T, D_IN)
    y = _moe(xf, Wr, br.reshape(1, E), We, be.reshape(E, 1, D_OUT))
    return y.reshape(x.shape[0], T, D_OUT)


# speedup vs baseline: 1.0032x; 1.0032x over previous
import jax
import jax.numpy as jnp
from jax import lax
from jax.experimental import pallas as pl
from jax.experimental.pallas import tpu as pltpu

D_IN = 768
D_OUT = 768
E = 8
T = 2048


def _moe_body(x_ref, wr_ref, br_ref, we_ref, be_ref, out_ref, mask_ref):
    e = pl.program_id(0)

    @pl.when(e == 0)
    def _router():
        logits = lax.dot_general(
            x_ref[...], wr_ref[...], (((1,), (1,)), ((), ())),
            preferred_element_type=jnp.float32,
        ) + br_ref[...]
        i1 = jnp.argmax(logits, axis=1)
        eids = lax.broadcasted_iota(jnp.int32, logits.shape, 1)
        m1 = eids == i1[:, None]
        l2 = jnp.where(m1, -jnp.inf, logits)
        i2 = jnp.argmax(l2, axis=1)
        m2 = eids == i2[:, None]
        mask_ref[...] = (m1 | m2).astype(jnp.float32)

    m = mask_ref[...]
    sel = (lax.broadcasted_iota(jnp.int32, m.shape, 1) == e).astype(jnp.float32)
    col = jnp.sum(m * sel, axis=1, keepdims=True)
    contrib = lax.dot_general(
        x_ref[...], we_ref[0], (((1,), (1,)), ((), ())),
        preferred_element_type=jnp.float32,
    ) + be_ref[0]
    contrib = col * contrib

    @pl.when(e == 0)
    def _init():
        out_ref[...] = contrib

    @pl.when(e != 0)
    def _acc():
        out_ref[...] += contrib


@jax.jit
def _moe(xf, Wr, br2, We, be3):
    return pl.pallas_call(
        _moe_body,
        grid=(E,),
        in_specs=[
            pl.BlockSpec((T, D_IN), lambda e: (0, 0)),
            pl.BlockSpec((E, D_IN), lambda e: (0, 0)),
            pl.BlockSpec((1, E), lambda e: (0, 0)),
            pl.BlockSpec((1, D_OUT, D_IN), lambda e: (e, 0, 0)),
            pl.BlockSpec((1, 1, D_OUT), lambda e: (e, 0, 0)),
        ],
        out_specs=pl.BlockSpec((T, D_OUT), lambda e: (0, 0)),
        out_shape=jax.ShapeDtypeStruct((T, D_OUT), jnp.float32),
        scratch_shapes=[pltpu.VMEM((T, E), jnp.float32)],
    )(xf, Wr, br2, We, be3)


def kernel(x, Wr, br, We, be):
    xf = x.reshape(T, D_IN)
    y = _moe(xf, Wr, br.reshape(1, E), We, be.reshape(E, 1, D_OUT))
    return y.reshape(x.shape[0], T, D_OUT)
